# Initial kernel scaffold; baseline (speedup 1.0000x reference)
import jax
import jax.numpy as jnp
from jax.experimental import pallas as pl


def kernel(x, embed, pos, Wqkv, bqkv, Wo, bo, ln1_g, ln1_b, W1, b1, W2, b2, ln2_g, ln2_b, cnt_ln_g, cnt_ln_b, Wc, bc, Wh, bh):
    raise NotImplementedError("write your pallas kernel here")


# speedup vs baseline: 2.0544x; 2.0544x over previous
import functools
import math

import jax
import jax.numpy as jnp
from jax import lax
from jax.experimental import pallas as pl
from jax.experimental.pallas import tpu as pltpu

_EPS = 1e-5
_NHEADS = 8
_INTERPRET = False


def _gather_kernel(x_smem, embed_any, tc_ref, h0_ref, s_ref, sem):
    per = pl.num_programs(1)
    b = pl.program_id(0) * per + pl.program_id(1)
    L = x_smem.shape[1]
    CD = tc_ref.shape[2]

    copies = []
    for t in range(L):
        tok = x_smem[b, t]
        cp = pltpu.make_async_copy(embed_any.at[tok], h0_ref.at[0, t], sem)
        cp.start()
        copies.append(cp)

    accs = [jnp.zeros((1, CD), jnp.float32) for _ in range(8)]
    for t in range(L):
        tok = x_smem[b, t]
        accs[t % 8] = accs[t % 8] + tc_ref[tok]
    acc = ((accs[0] + accs[1]) + (accs[2] + accs[3])) + (
        (accs[4] + accs[5]) + (accs[6] + accs[7]))
    s_ref[0] = acc

    for cp in copies:
        cp.wait()


def _ln(v, g, b):
    m = jnp.mean(v, axis=-1, keepdims=True)
    xc = v - m
    var = jnp.mean(xc * xc, axis=-1, keepdims=True)
    return xc * lax.rsqrt(var + _EPS) * g + b


def _gelu_exact(v):
    return 0.5 * v * (1.0 + lax.erf(v * (1.0 / math.sqrt(2.0))))


def _encoder_kernel(V, h0_any, pos_ref, x_ref, s_in_ref,
                    wqkv_ref, bqkv_ref, wo_ref, bo_ref, ln1g_ref, ln1b_ref,
                    w1_ref, b1_ref, w2_ref, b2_ref, ln2g_ref, ln2b_ref,
                    tc0_ref, g_ref, bc2_ref, wht_ref, bh_ref,
                    out_ref, h_s, qkv_s, sc_s, o_s, ff_s, sem):
    c = pl.program_id(0)
    l = pl.program_id(1)
    NL = pl.num_programs(1)
    S_half, L = x_ref.shape
    D = pos_ref.shape[1]
    DH = D // _NHEADS
    CD = s_in_ref.shape[2]
    bf = jnp.bfloat16
    f32 = jnp.float32
    scale = 1.0 / math.sqrt(DH)

    @pl.when(l == 0)
    def _init():
        cp = pltpu.make_async_copy(h0_any.at[c], h_s, sem)
        cp.start()
        cp.wait()
        for s in range(S_half):
            h_s[s * L:(s + 1) * L, :] = h_s[s * L:(s + 1) * L, :] + pos_ref[...]

    wqkv = wqkv_ref[0]
    wo = wo_ref[0]
    w1 = w1_ref[0]
    w2 = w2_ref[0]

    def seq_body(s, carry):
        off = pl.multiple_of(s * L, L)
        hs = h_s[pl.ds(off, L), :]
        qkv_s[...] = jnp.dot(hs.astype(bf), wqkv,
                             preferred_element_type=f32) + bqkv_ref[0]
        for hh in range(_NHEADS):
            q = qkv_s[:, hh * DH:(hh + 1) * DH].astype(bf)
            k = qkv_s[:, D + hh * DH:D + (hh + 1) * DH].astype(bf)
            sc_s[...] = lax.dot_general(
                q, k, (((1,), (1,)), ((), ())),
                preferred_element_type=f32) * scale
            for rc in range(L // 128):
                blk = sc_s[rc * 128:(rc + 1) * 128, :]
                mx = jnp.max(blk, axis=-1, keepdims=True)
                e = jnp.exp(blk - mx)
                sm = jnp.sum(e, axis=-1, keepdims=True)
                sc_s[rc * 128:(rc + 1) * 128, :] = e * (1.0 / sm)
            v = qkv_s[:, 2 * D + hh * DH:2 * D + (hh + 1) * DH].astype(bf)
            o_s[:, hh * DH:(hh + 1) * DH] = lax.dot_general(
                sc_s[...].astype(bf), v, (((1,), (0,)), ((), ())),
                preferred_element_type=f32)
        attn = jnp.dot(o_s[...].astype(bf), wo,
                       preferred_element_type=f32) + bo_ref[0]
        h1 = _ln(hs + attn, ln1g_ref[0], ln1b_ref[0])
        ff_s[...] = jnp.dot(h1.astype(bf), w1,
                            preferred_element_type=f32) + b1_ref[0]
        for cc in range((4 * D) // 512):
            blk = ff_s[:, cc * 512:(cc + 1) * 512]
            ff_s[:, cc * 512:(cc + 1) * 512] = _gelu_exact(blk)
        ff2 = jnp.dot(ff_s[...].astype(bf), w2,
                      preferred_element_type=f32) + b2_ref[0]
        h_s[pl.ds(off, L), :] = _ln(h1 + ff2, ln2g_ref[0], ln2b_ref[0])
        return carry

    lax.fori_loop(0, S_half, seq_body, 0)

    @pl.when(l == NL - 1)
    def _head():
        ctxs = []
        for s in range(S_half):
            hs = h_s[s * L:(s + 1) * L, :]
            ctxs.append(jnp.mean(hs, axis=0, keepdims=True))
        ctx = jnp.concatenate(ctxs, axis=0)

        xb = x_ref[...]
        iota = lax.broadcasted_iota(jnp.int32, xb.shape, 1)
        xs = jnp.where(xb == 0, -(iota + 1), xb)
        nb = jnp.sum((xb != 0).astype(f32), axis=-1, keepdims=True)
        npad = float(L) - nb
        cnt = jnp.zeros(xs.shape, f32)
        for t in range(L):
            col = xs[:, t:t + 1]
            cnt = cnt + (xs == col).astype(f32)
        pair = jnp.sum(cnt, axis=-1, keepdims=True)
        sumsq = pair - npad
        m = nb * (1.0 / V)
        var = sumsq * (1.0 / V) - m * m
        r = lax.rsqrt(var + _EPS)
        S = s_in_ref[...].reshape(S_half, CD) - npad * tc0_ref[...]
        cf = jnp.maximum(r * S - (r * m) * g_ref[...] + bc2_ref[...], 0.0)
        feat = jnp.concatenate([ctx, cf], axis=-1)
        out_ref[...] = jnp.dot(feat, wht_ref[...],
                               preferred_element_type=f32) + bh_ref[...]


def kernel(x, embed, pos, Wqkv, bqkv, Wo, bo, ln1_g, ln1_b, W1, b1, W2, b2,
           ln2_g, ln2_b, cnt_ln_g, cnt_ln_b, Wc, bc, Wh, bh):
    B, L = x.shape
    V, D = embed.shape
    NL = Wqkv.shape[0]
    CD = Wc.shape[0]
    C = Wh.shape[0]
    S_half = B // 2
    bf = jnp.bfloat16
    f32 = jnp.float32

    wqkvT = Wqkv.transpose(0, 2, 1).astype(bf)
    woT = Wo.transpose(0, 2, 1).astype(bf)
    w1T = W1.transpose(0, 2, 1).astype(bf)
    w2T = W2.transpose(0, 2, 1).astype(bf)
    bqkv2 = bqkv[:, None, :]
    bo2 = bo[:, None, :]
    b12 = b1[:, None, :]
    b22 = b2[:, None, :]
    ln1g2 = ln1_g[:, None, :]
    ln1b2 = ln1_b[:, None, :]
    ln2g2 = ln2_g[:, None, :]
    ln2b2 = ln2_b[:, None, :]
    pos2 = pos[0]
    embed3 = embed[:, None, :]
    Tc = (Wc * cnt_ln_g[None, :]).T[:, None, :]
    tc0 = (Wc[:, 0] * cnt_ln_g[0])[None, :]
    G2 = (Wc @ cnt_ln_g)[None, :]
    Bc2 = (Wc @ cnt_ln_b + bc)[None, :]
    whT = Wh.T
    bh2 = bh[None, :]

    h0, S = pl.pallas_call(
        _gather_kernel,
        grid=(2, S_half),
        in_specs=[
            pl.BlockSpec(memory_space=pltpu.SMEM),
            pl.BlockSpec(memory_space=pl.ANY),
            pl.BlockSpec((V, 1, CD), lambda c, i: (0, 0, 0)),
        ],
        out_specs=[
            pl.BlockSpec((1, L, 1, D), lambda c, i, _per=S_half: (c * _per + i, 0, 0, 0)),
            pl.BlockSpec((1, 1, CD), lambda c, i, _per=S_half: (c * _per + i, 0, 0)),
        ],
        out_shape=[
            jax.ShapeDtypeStruct((B, L, 1, D), f32),
            jax.ShapeDtypeStruct((B, 1, CD), f32),
        ],
        scratch_shapes=[pltpu.SemaphoreType.DMA],
        compiler_params=pltpu.CompilerParams(
            dimension_semantics=("parallel", "arbitrary"),
            vmem_limit_bytes=44 * 1024 * 1024,
        ),
        name="embed_gather",
        interpret=_INTERPRET,
    )(x, embed3, Tc)

    h0r = h0.reshape(2, S_half * L, D)

    logits = pl.pallas_call(
        functools.partial(_encoder_kernel, float(V)),
        grid=(2, NL),
        in_specs=[
            pl.BlockSpec(memory_space=pl.ANY),
            pl.BlockSpec((L, D), lambda c, l: (0, 0)),
            pl.BlockSpec((S_half, L), lambda c, l: (c, 0)),
            pl.BlockSpec((S_half, 1, CD), lambda c, l: (c, 0, 0)),
            pl.BlockSpec((1, D, 3 * D), lambda c, l: (l, 0, 0)),
            pl.BlockSpec((1, 1, 3 * D), lambda c, l: (l, 0, 0)),
            pl.BlockSpec((1, D, D), lambda c, l: (l, 0, 0)),
            pl.BlockSpec((1, 1, D), lambda c, l: (l, 0, 0)),
            pl.BlockSpec((1, 1, D), lambda c, l: (l, 0, 0)),
            pl.BlockSpec((1, 1, D), lambda c, l: (l, 0, 0)),
            pl.BlockSpec((1, D, 4 * D), lambda c, l: (l, 0, 0)),
            pl.BlockSpec((1, 1, 4 * D), lambda c, l: (l, 0, 0)),
            pl.BlockSpec((1, 4 * D, D), lambda c, l: (l, 0, 0)),
            pl.BlockSpec((1, 1, D), lambda c, l: (l, 0, 0)),
            pl.BlockSpec((1, 1, D), lambda c, l: (l, 0, 0)),
            pl.BlockSpec((1, 1, D), lambda c, l: (l, 0, 0)),
            pl.BlockSpec((1, CD), lambda c, l: (0, 0)),
            pl.BlockSpec((1, CD), lambda c, l: (0, 0)),
            pl.BlockSpec((1, CD), lambda c, l: (0, 0)),
            pl.BlockSpec((D + CD, C), lambda c, l: (0, 0)),
            pl.BlockSpec((1, C), lambda c, l: (0, 0)),
        ],
        out_specs=pl.BlockSpec((S_half, C), lambda c, l: (c, 0)),
        out_shape=jax.ShapeDtypeStruct((B, C), f32),
        scratch_shapes=[
            pltpu.VMEM((S_half * L, D), f32),
            pltpu.VMEM((L, 3 * D), f32),
            pltpu.VMEM((L, L), f32),
            pltpu.VMEM((L, D), f32),
            pltpu.VMEM((L, 4 * D), f32),
            pltpu.SemaphoreType.DMA,
        ],
        compiler_params=pltpu.CompilerParams(
            dimension_semantics=("parallel", "arbitrary"),
            vmem_limit_bytes=50 * 1024 * 1024,
        ),
        name="encoder_head",
        interpret=_INTERPRET,
    )(h0r, pos2, x, S, wqkvT, bqkv2, woT, bo2, ln1g2, ln1b2,
      w1T, b12, w2T, b22, ln2g2, ln2b2, tc0, G2, Bc2, whT, bh2)

    return logits


# speedup vs baseline: 2.6762x; 1.3027x over previous
import functools
import math

import jax
import jax.numpy as jnp
from jax import lax
from jax.experimental import pallas as pl
from jax.experimental.pallas import tpu as pltpu

_EPS = 1e-5
_NHEADS = 8
_INTERPRET = False


def _gather_kernel(NB, x_smem, embed_any, wc_ref, g_ref, b_ref,
                   h0_ref, s_ref, gacc_ref, bacc_ref, tc_s, sem):
    c = pl.program_id(0)
    i = pl.program_id(1)
    per = pl.num_programs(1) - NB
    L = x_smem.shape[1]
    VB = wc_ref.shape[1]
    CD = wc_ref.shape[0]

    @pl.when(i < NB)
    def _build():
        t = jnp.transpose(wc_ref[...])
        tc = t * g_ref[...]
        tc_s[pl.ds(i * VB, VB)] = tc.reshape(VB, 1, CD)
        gp = jnp.sum(tc, axis=0, keepdims=True)
        bp = jnp.sum(t * b_ref[...], axis=0, keepdims=True)

        @pl.when(i == 0)
        def _():
            gacc_ref[...] = jnp.zeros_like(gacc_ref)
            bacc_ref[...] = jnp.zeros_like(bacc_ref)

        gacc_ref[...] += gp
        bacc_ref[...] += bp

    @pl.when(i >= NB)
    def _gather():
        b = c * per + (i - NB)
        copies = []
        for t in range(L):
            tok = x_smem[b, t]
            cp = pltpu.make_async_copy(
                embed_any.at[pl.ds(tok, 1), :], h0_ref.at[0, t], sem)
            cp.start()
            copies.append(cp)

        accs = [jnp.zeros((1, CD), jnp.float32) for _ in range(8)]
        npad = jnp.int32(0)
        for t in range(L):
            tok = x_smem[b, t]
            accs[t % 8] = accs[t % 8] + tc_s[tok]
            npad = npad + jnp.where(tok == 0, 1, 0).astype(jnp.int32)
        acc = ((accs[0] + accs[1]) + (accs[2] + accs[3])) + (
            (accs[4] + accs[5]) + (accs[6] + accs[7]))
        s_ref[0] = acc - npad.astype(jnp.float32) * tc_s[0]

        for cp in copies:
            cp.wait()


def _ln(v, g, b):
    m = jnp.mean(v, axis=-1, keepdims=True)
    xc = v - m
    var = jnp.mean(xc * xc, axis=-1, keepdims=True)
    return xc * lax.rsqrt(var + _EPS) * g + b


def _gelu_exact(v):
    return 0.5 * v * (1.0 + lax.erf(v * (1.0 / math.sqrt(2.0))))


def _encoder_kernel(V, h0_any, pos_ref, x_ref, s_in_ref,
                    wqkv_ref, bqkv_ref, wo_ref, bo_ref, ln1g_ref, ln1b_ref,
                    w1_ref, b1_ref, w2_ref, b2_ref, ln2g_ref, ln2b_ref,
                    g_ref, bc2_ref, wht_ref, bh_ref,
                    out_ref, h_s, qkv_s, sc_s, o_s, ff_s, sem):
    c = pl.program_id(0)
    l = pl.program_id(1)
    NL = pl.num_programs(1)
    S_half, L = x_ref.shape
    D = pos_ref.shape[1]
    DH = D // _NHEADS
    CD = s_in_ref.shape[2]
    bf = jnp.bfloat16
    f32 = jnp.float32
    scale = 1.0 / math.sqrt(DH)

    @pl.when(l == 0)
    def _init():
        cp = pltpu.make_async_copy(h0_any.at[c], h_s, sem)
        cp.start()
        cp.wait()
        for s in range(S_half):
            h_s[s * L:(s + 1) * L, :] = h_s[s * L:(s + 1) * L, :] + pos_ref[...]

    wqkv = wqkv_ref[0]
    wo = wo_ref[0]
    w1 = w1_ref[0]
    w2 = w2_ref[0]

    def seq_body(s, carry):
        off = pl.multiple_of(s * L, L)
        hs = h_s[pl.ds(off, L), :]
        qkv_s[...] = lax.dot_general(
            hs.astype(bf), wqkv, (((1,), (1,)), ((), ())),
            preferred_element_type=f32) + bqkv_ref[0]
        for hh in range(_NHEADS):
            q = qkv_s[:, hh * DH:(hh + 1) * DH].astype(bf)
            k = qkv_s[:, D + hh * DH:D + (hh + 1) * DH].astype(bf)
            sc_s[...] = lax.dot_general(
                q, k, (((1,), (1,)), ((), ())),
                preferred_element_type=f32) * scale
            for rc in range(L // 128):
                blk = sc_s[rc * 128:(rc + 1) * 128, :]
                mx = jnp.max(blk, axis=-1, keepdims=True)
                e = jnp.exp(blk - mx)
                sm = jnp.sum(e, axis=-1, keepdims=True)
                sc_s[rc * 128:(rc + 1) * 128, :] = e * (1.0 / sm)
            v = qkv_s[:, 2 * D + hh * DH:2 * D + (hh + 1) * DH].astype(bf)
            o_s[:, hh * DH:(hh + 1) * DH] = lax.dot_general(
                sc_s[...].astype(bf), v, (((1,), (0,)), ((), ())),
                preferred_element_type=f32)
        attn = lax.dot_general(
            o_s[...].astype(bf), wo, (((1,), (1,)), ((), ())),
            preferred_element_type=f32) + bo_ref[0]
        h1 = _ln(hs + attn, ln1g_ref[0], ln1b_ref[0])
        ff_s[...] = lax.dot_general(
            h1.astype(bf), w1, (((1,), (1,)), ((), ())),
            preferred_element_type=f32) + b1_ref[0]
        for cc in range((4 * D) // 512):
            blk = ff_s[:, cc * 512:(cc + 1) * 512]
            ff_s[:, cc * 512:(cc + 1) * 512] = _gelu_exact(blk)
        ff2 = lax.dot_general(
            ff_s[...].astype(bf), w2, (((1,), (1,)), ((), ())),
            preferred_element_type=f32) + b2_ref[0]
        h_s[pl.ds(off, L), :] = _ln(h1 + ff2, ln2g_ref[0], ln2b_ref[0])
        return carry

    lax.fori_loop(0, S_half, seq_body, 0)

    @pl.when(l == NL - 1)
    def _head():
        ctxs = []
        for s in range(S_half):
            hs = h_s[s * L:(s + 1) * L, :]
            ctxs.append(jnp.mean(hs, axis=0, keepdims=True))
        ctx = jnp.concatenate(ctxs, axis=0)

        xb = x_ref[...]
        iota = lax.broadcasted_iota(jnp.int32, xb.shape, 1)
        xs = jnp.where(xb == 0, -(iota + 1), xb)
        nb = jnp.sum((xb != 0).astype(f32), axis=-1, keepdims=True)
        npad = float(L) - nb
        cnt = jnp.zeros(xs.shape, f32)
        for t in range(L):
            col = xs[:, t:t + 1]
            cnt = cnt + (xs == col).astype(f32)
        pair = jnp.sum(cnt, axis=-1, keepdims=True)
        sumsq = pair - npad
        m = nb * (1.0 / V)
        var = sumsq * (1.0 / V) - m * m
        r = lax.rsqrt(var + _EPS)
        S = s_in_ref[...].reshape(S_half, CD)
        cf = jnp.maximum(r * S - (r * m) * g_ref[...] + bc2_ref[...], 0.0)
        feat = jnp.concatenate([ctx, cf], axis=-1)
        out_ref[...] = jnp.dot(feat, wht_ref[...],
                               preferred_element_type=f32) + bh_ref[...]


def kernel(x, embed, pos, Wqkv, bqkv, Wo, bo, ln1_g, ln1_b, W1, b1, W2, b2,
           ln2_g, ln2_b, cnt_ln_g, cnt_ln_b, Wc, bc, Wh, bh):
    B, L = x.shape
    V, D = embed.shape
    NL = Wqkv.shape[0]
    CD = Wc.shape[0]
    C = Wh.shape[0]
    S_half = B // 2
    bf = jnp.bfloat16
    f32 = jnp.float32

    wqkvT = Wqkv.astype(bf)
    woT = Wo.astype(bf)
    w1T = W1.astype(bf)
    w2T = W2.astype(bf)
    bqkv2 = bqkv[:, None, :]
    bo2 = bo[:, None, :]
    b12 = b1[:, None, :]
    b22 = b2[:, None, :]
    ln1g2 = ln1_g[:, None, :]
    ln1b2 = ln1_b[:, None, :]
    ln2g2 = ln2_g[:, None, :]
    ln2b2 = ln2_b[:, None, :]
    pos2 = pos[0]
    whT = Wh.T
    bh2 = bh[None, :]
    VB = 4096
    NB = -(-V // VB)
    Vp = NB * VB
    Wcp = jnp.pad(Wc, ((0, 0), (0, Vp - V)))
    gp3 = jnp.pad(cnt_ln_g, (0, Vp - V)).reshape(Vp, 1)
    bp3 = jnp.pad(cnt_ln_b, (0, Vp - V)).reshape(Vp, 1)
    h0, S, G2, Bc0 = pl.pallas_call(
        functools.partial(_gather_kernel, NB),
        grid=(2, NB + S_half),
        in_specs=[
            pl.BlockSpec(memory_space=pltpu.SMEM),
            pl.BlockSpec(memory_space=pl.ANY),
            pl.BlockSpec((CD, VB), lambda c, i, _nb=NB: (0, jnp.minimum(i, _nb - 1))),
            pl.BlockSpec((VB, 1), lambda c, i, _nb=NB: (jnp.minimum(i, _nb - 1), 0)),
            pl.BlockSpec((VB, 1), lambda c, i, _nb=NB: (jnp.minimum(i, _nb - 1), 0)),
        ],
        out_specs=[
            pl.BlockSpec((1, L, 1, D), lambda c, i, _per=S_half, _nb=NB:
                         (c * _per + jnp.maximum(i - _nb, 0), 0, 0, 0)),
            pl.BlockSpec((1, 1, CD), lambda c, i, _per=S_half, _nb=NB:
                         (c * _per + jnp.maximum(i - _nb, 0), 0, 0)),
            pl.BlockSpec((1, CD), lambda c, i: (0, 0)),
            pl.BlockSpec((1, CD), lambda c, i: (0, 0)),
        ],
        out_shape=[
            jax.ShapeDtypeStruct((B, L, 1, D), f32),
            jax.ShapeDtypeStruct((B, 1, CD), f32),
            jax.ShapeDtypeStruct((1, CD), f32),
            jax.ShapeDtypeStruct((1, CD), f32),
        ],
        scratch_shapes=[
            pltpu.VMEM((Vp, 1, CD), f32),
            pltpu.SemaphoreType.DMA,
        ],
        compiler_params=pltpu.CompilerParams(
            dimension_semantics=("parallel", "arbitrary"),
            vmem_limit_bytes=44 * 1024 * 1024,
        ),
        name="table_embed_gather",
        interpret=_INTERPRET,
    )(x, embed, Wcp, gp3, bp3)
    Bc2 = Bc0 + bc[None, :]

    h0r = h0.reshape(2, S_half * L, D)

    logits = pl.pallas_call(
        functools.partial(_encoder_kernel, float(V)),
        grid=(2, NL),
        in_specs=[
            pl.BlockSpec(memory_space=pl.ANY),
            pl.BlockSpec((L, D), lambda c, l: (0, 0)),
            pl.BlockSpec((S_half, L), lambda c, l: (c, 0)),
            pl.BlockSpec((S_half, 1, CD), lambda c, l: (c, 0, 0)),
            pl.BlockSpec((1, 3 * D, D), lambda c, l: (l, 0, 0)),
            pl.BlockSpec((1, 1, 3 * D), lambda c, l: (l, 0, 0)),
            pl.BlockSpec((1, D, D), lambda c, l: (l, 0, 0)),
            pl.BlockSpec((1, 1, D), lambda c, l: (l, 0, 0)),
            pl.BlockSpec((1, 1, D), lambda c, l: (l, 0, 0)),
            pl.BlockSpec((1, 1, D), lambda c, l: (l, 0, 0)),
            pl.BlockSpec((1, 4 * D, D), lambda c, l: (l, 0, 0)),
            pl.BlockSpec((1, 1, 4 * D), lambda c, l: (l, 0, 0)),
            pl.BlockSpec((1, D, 4 * D), lambda c, l: (l, 0, 0)),
            pl.BlockSpec((1, 1, D), lambda c, l: (l, 0, 0)),
            pl.BlockSpec((1, 1, D), lambda c, l: (l, 0, 0)),
            pl.BlockSpec((1, 1, D), lambda c, l: (l, 0, 0)),
            pl.BlockSpec((1, CD), lambda c, l: (0, 0)),
            pl.BlockSpec((1, CD), lambda c, l: (0, 0)),
            pl.BlockSpec((D + CD, C), lambda c, l: (0, 0)),
            pl.BlockSpec((1, C), lambda c, l: (0, 0)),
        ],
        out_specs=pl.BlockSpec((S_half, C), lambda c, l: (c, 0)),
        out_shape=jax.ShapeDtypeStruct((B, C), f32),
        scratch_shapes=[
            pltpu.VMEM((S_half * L, D), f32),
            pltpu.VMEM((L, 3 * D), f32),
            pltpu.VMEM((L, L), f32),
            pltpu.VMEM((L, D), f32),
            pltpu.VMEM((L, 4 * D), f32),
            pltpu.SemaphoreType.DMA,
        ],
        compiler_params=pltpu.CompilerParams(
            dimension_semantics=("parallel", "arbitrary"),
            vmem_limit_bytes=50 * 1024 * 1024,
        ),
        name="encoder_head",
        interpret=_INTERPRET,
    )(h0r, pos2, x, S, wqkvT, bqkv2, woT, bo2, ln1g2, ln1b2,
      w1T, b12, w2T, b22, ln2g2, ln2b2, G2, Bc2, whT, bh2)

    return logits


# speedup vs baseline: 2.7167x; 1.0151x over previous
import functools
import math

import jax
import jax.numpy as jnp
from jax import lax
from jax.experimental import pallas as pl
from jax.experimental.pallas import tpu as pltpu

_EPS = 1e-5
_NHEADS = 8
_INTERPRET = False


def _gather_kernel(NB, x_smem, embed_any, wc_ref, g_ref, b_ref,
                   h0_ref, s_ref, gacc_ref, bacc_ref, tc_s, sem):
    c = pl.program_id(0)
    i = pl.program_id(1)
    per = pl.num_programs(1) - NB
    L = x_smem.shape[1]
    VB = wc_ref.shape[1]
    CD = wc_ref.shape[0]

    @pl.when((i < NB) & (c == 0))
    def _build():
        t = jnp.transpose(wc_ref[...])
        tc = t * g_ref[...]
        tc_s[pl.ds(i * VB, VB)] = tc.reshape(VB, 1, CD)
        gp = jnp.sum(tc, axis=0, keepdims=True)
        bp = jnp.sum(t * b_ref[...], axis=0, keepdims=True)

        @pl.when(i == 0)
        def _():
            gacc_ref[...] = jnp.zeros_like(gacc_ref)
            bacc_ref[...] = jnp.zeros_like(bacc_ref)

        gacc_ref[...] += gp
        bacc_ref[...] += bp

    @pl.when(i >= NB)
    def _gather():
        b = c * per + (i - NB)
        copies = []
        for t in range(L):
            tok = x_smem[b, t]
            cp = pltpu.make_async_copy(
                embed_any.at[pl.ds(tok, 1), :], h0_ref.at[0, t], sem)
            cp.start()
            copies.append(cp)

        accs = [jnp.zeros((1, CD), jnp.float32) for _ in range(8)]
        npad = jnp.int32(0)
        for t in range(L):
            tok = x_smem[b, t]
            accs[t % 8] = accs[t % 8] + tc_s[tok]
            npad = npad + jnp.where(tok == 0, 1, 0).astype(jnp.int32)
        acc = ((accs[0] + accs[1]) + (accs[2] + accs[3])) + (
            (accs[4] + accs[5]) + (accs[6] + accs[7]))
        s_ref[0] = acc - npad.astype(jnp.float32) * tc_s[0]

        for cp in copies:
            cp.wait()


def _ln(v, g, b):
    m = jnp.mean(v, axis=-1, keepdims=True)
    xc = v - m
    var = jnp.mean(xc * xc, axis=-1, keepdims=True)
    return xc * lax.rsqrt(var + _EPS) * g + b


def _gelu_exact(v):
    return 0.5 * v * (1.0 + lax.erf(v * (1.0 / math.sqrt(2.0))))


def _encoder_kernel(V, h0_any, pos_ref, x_ref, s_in_ref,
                    wqkv_ref, bqkv_ref, wo_ref, bo_ref, ln1g_ref, ln1b_ref,
                    w1_ref, b1_ref, w2_ref, b2_ref, ln2g_ref, ln2b_ref,
                    g_ref, bc2_ref, wht_ref, bh_ref,
                    out_ref, h_s, qkv_s, sc_s, o_s, ff_s, sem):
    c = pl.program_id(0)
    l = pl.program_id(1)
    NL = pl.num_programs(1)
    S_half, L = x_ref.shape
    D = pos_ref.shape[1]
    DH = D // _NHEADS
    CD = s_in_ref.shape[2]
    bf = jnp.bfloat16
    f32 = jnp.float32
    scale = 1.0 / math.sqrt(DH)

    @pl.when(l == 0)
    def _init():
        cp = pltpu.make_async_copy(h0_any.at[c], h_s, sem)
        cp.start()
        cp.wait()
        for s in range(S_half):
            h_s[s * L:(s + 1) * L, :] = h_s[s * L:(s + 1) * L, :] + pos_ref[...]

    wqkv = wqkv_ref[0]
    wo = wo_ref[0]
    w1 = w1_ref[0]
    w2 = w2_ref[0]

    def seq_body(s, carry):
        off = pl.multiple_of(s * L, L)
        hs = h_s[pl.ds(off, L), :]
        qkv_s[...] = lax.dot_general(
            hs.astype(bf), wqkv, (((1,), (1,)), ((), ())),
            preferred_element_type=f32) + bqkv_ref[0]
        for hh in range(_NHEADS):
            q = qkv_s[:, hh * DH:(hh + 1) * DH].astype(bf)
            k = qkv_s[:, D + hh * DH:D + (hh + 1) * DH].astype(bf)
            sc_s[...] = lax.dot_general(
                q, k, (((1,), (1,)), ((), ())),
                preferred_element_type=f32) * scale
            for rc in range(L // 128):
                blk = sc_s[rc * 128:(rc + 1) * 128, :]
                mx = jnp.max(blk, axis=-1, keepdims=True)
                e = jnp.exp(blk - mx)
                sm = jnp.sum(e, axis=-1, keepdims=True)
                sc_s[rc * 128:(rc + 1) * 128, :] = e * (1.0 / sm)
            v = qkv_s[:, 2 * D + hh * DH:2 * D + (hh + 1) * DH].astype(bf)
            o_s[:, hh * DH:(hh + 1) * DH] = lax.dot_general(
                sc_s[...].astype(bf), v, (((1,), (0,)), ((), ())),
                preferred_element_type=f32)
        attn = lax.dot_general(
            o_s[...].astype(bf), wo, (((1,), (1,)), ((), ())),
            preferred_element_type=f32) + bo_ref[0]
        h1 = _ln(hs + attn, ln1g_ref[0], ln1b_ref[0])
        ff_s[...] = lax.dot_general(
            h1.astype(bf), w1, (((1,), (1,)), ((), ())),
            preferred_element_type=f32) + b1_ref[0]
        for cc in range((4 * D) // 512):
            blk = ff_s[:, cc * 512:(cc + 1) * 512]
            ff_s[:, cc * 512:(cc + 1) * 512] = _gelu_exact(blk)
        ff2 = lax.dot_general(
            ff_s[...].astype(bf), w2, (((1,), (1,)), ((), ())),
            preferred_element_type=f32) + b2_ref[0]
        h_s[pl.ds(off, L), :] = _ln(h1 + ff2, ln2g_ref[0], ln2b_ref[0])
        return carry

    lax.fori_loop(0, S_half, seq_body, 0)

    @pl.when(l == NL - 1)
    def _head():
        ctxs = []
        for s in range(S_half):
            hs = h_s[s * L:(s + 1) * L, :]
            ctxs.append(jnp.mean(hs, axis=0, keepdims=True))
        ctx = jnp.concatenate(ctxs, axis=0)

        xb = x_ref[...]
        iota = lax.broadcasted_iota(jnp.int32, xb.shape, 1)
        xs = jnp.where(xb == 0, -(iota + 1), xb)
        nb = jnp.sum((xb != 0).astype(f32), axis=-1, keepdims=True)
        npad = float(L) - nb
        cnt = jnp.zeros(xs.shape, f32)
        for t in range(L):
            col = xs[:, t:t + 1]
            cnt = cnt + (xs == col).astype(f32)
        pair = jnp.sum(cnt, axis=-1, keepdims=True)
        sumsq = pair - npad
        m = nb * (1.0 / V)
        var = sumsq * (1.0 / V) - m * m
        r = lax.rsqrt(var + _EPS)
        S = s_in_ref[...].reshape(S_half, CD)
        cf = jnp.maximum(r * S - (r * m) * g_ref[...] + bc2_ref[...], 0.0)
        feat = jnp.concatenate([ctx, cf], axis=-1)
        out_ref[...] = jnp.dot(feat, wht_ref[...],
                               preferred_element_type=f32) + bh_ref[...]


def kernel(x, embed, pos, Wqkv, bqkv, Wo, bo, ln1_g, ln1_b, W1, b1, W2, b2,
           ln2_g, ln2_b, cnt_ln_g, cnt_ln_b, Wc, bc, Wh, bh):
    B, L = x.shape
    V, D = embed.shape
    NL = Wqkv.shape[0]
    CD = Wc.shape[0]
    C = Wh.shape[0]
    S_half = B // 2
    bf = jnp.bfloat16
    f32 = jnp.float32

    wqkvT = Wqkv.astype(bf)
    woT = Wo.astype(bf)
    w1T = W1.astype(bf)
    w2T = W2.astype(bf)
    bqkv2 = bqkv[:, None, :]
    bo2 = bo[:, None, :]
    b12 = b1[:, None, :]
    b22 = b2[:, None, :]
    ln1g2 = ln1_g[:, None, :]
    ln1b2 = ln1_b[:, None, :]
    ln2g2 = ln2_g[:, None, :]
    ln2b2 = ln2_b[:, None, :]
    pos2 = pos[0]
    whT = Wh.T
    bh2 = bh[None, :]
    VB = 4096
    NB = -(-V // VB)
    Vp = NB * VB
    Wcp = jnp.pad(Wc, ((0, 0), (0, Vp - V)))
    gp3 = jnp.pad(cnt_ln_g, (0, Vp - V)).reshape(Vp, 1)
    bp3 = jnp.pad(cnt_ln_b, (0, Vp - V)).reshape(Vp, 1)
    h0, S, G2, Bc0 = pl.pallas_call(
        functools.partial(_gather_kernel, NB),
        grid=(2, NB + S_half),
        in_specs=[
            pl.BlockSpec(memory_space=pltpu.SMEM),
            pl.BlockSpec(memory_space=pl.ANY),
            pl.BlockSpec((CD, VB), lambda c, i, _nb=NB: (0, jnp.minimum(i, _nb - 1))),
            pl.BlockSpec((VB, 1), lambda c, i, _nb=NB: (jnp.minimum(i, _nb - 1), 0)),
            pl.BlockSpec((VB, 1), lambda c, i, _nb=NB: (jnp.minimum(i, _nb - 1), 0)),
        ],
        out_specs=[
            pl.BlockSpec((1, L, 1, D), lambda c, i, _per=S_half, _nb=NB:
                         (c * _per + jnp.maximum(i - _nb, 0), 0, 0, 0)),
            pl.BlockSpec((1, 1, CD), lambda c, i, _per=S_half, _nb=NB:
                         (c * _per + jnp.maximum(i - _nb, 0), 0, 0)),
            pl.BlockSpec((1, CD), lambda c, i: (0, 0)),
            pl.BlockSpec((1, CD), lambda c, i: (0, 0)),
        ],
        out_shape=[
            jax.ShapeDtypeStruct((B, L, 1, D), f32),
            jax.ShapeDtypeStruct((B, 1, CD), f32),
            jax.ShapeDtypeStruct((1, CD), f32),
            jax.ShapeDtypeStruct((1, CD), f32),
        ],
        scratch_shapes=[
            pltpu.VMEM((Vp, 1, CD), f32),
            pltpu.SemaphoreType.DMA,
        ],
        compiler_params=pltpu.CompilerParams(
            dimension_semantics=("parallel", "arbitrary"),
            vmem_limit_bytes=44 * 1024 * 1024,
        ),
        name="table_embed_gather",
        interpret=_INTERPRET,
    )(x, embed, Wcp, gp3, bp3)
    Bc2 = Bc0 + bc[None, :]

    h0r = h0.reshape(2, S_half * L, D)

    logits = pl.pallas_call(
        functools.partial(_encoder_kernel, float(V)),
        grid=(2, NL),
        in_specs=[
            pl.BlockSpec(memory_space=pl.ANY),
            pl.BlockSpec((L, D), lambda c, l: (0, 0)),
            pl.BlockSpec((S_half, L), lambda c, l: (c, 0)),
            pl.BlockSpec((S_half, 1, CD), lambda c, l: (c, 0, 0)),
            pl.BlockSpec((1, 3 * D, D), lambda c, l: (l, 0, 0)),
            pl.BlockSpec((1, 1, 3 * D), lambda c, l: (l, 0, 0)),
            pl.BlockSpec((1, D, D), lambda c, l: (l, 0, 0)),
            pl.BlockSpec((1, 1, D), lambda c, l: (l, 0, 0)),
            pl.BlockSpec((1, 1, D), lambda c, l: (l, 0, 0)),
            pl.BlockSpec((1, 1, D), lambda c, l: (l, 0, 0)),
            pl.BlockSpec((1, 4 * D, D), lambda c, l: (l, 0, 0)),
            pl.BlockSpec((1, 1, 4 * D), lambda c, l: (l, 0, 0)),
            pl.BlockSpec((1, D, 4 * D), lambda c, l: (l, 0, 0)),
            pl.BlockSpec((1, 1, D), lambda c, l: (l, 0, 0)),
            pl.BlockSpec((1, 1, D), lambda c, l: (l, 0, 0)),
            pl.BlockSpec((1, 1, D), lambda c, l: (l, 0, 0)),
            pl.BlockSpec((1, CD), lambda c, l: (0, 0)),
            pl.BlockSpec((1, CD), lambda c, l: (0, 0)),
            pl.BlockSpec((D + CD, C), lambda c, l: (0, 0)),
            pl.BlockSpec((1, C), lambda c, l: (0, 0)),
        ],
        out_specs=pl.BlockSpec((S_half, C), lambda c, l: (c, 0)),
        out_shape=jax.ShapeDtypeStruct((B, C), f32),
        scratch_shapes=[
            pltpu.VMEM((S_half * L, D), f32),
            pltpu.VMEM((L, 3 * D), f32),
            pltpu.VMEM((L, L), f32),
            pltpu.VMEM((L, D), f32),
            pltpu.VMEM((L, 4 * D), f32),
            pltpu.SemaphoreType.DMA,
        ],
        compiler_params=pltpu.CompilerParams(
            dimension_semantics=("parallel", "arbitrary"),
            vmem_limit_bytes=50 * 1024 * 1024,
        ),
        name="encoder_head",
        interpret=_INTERPRET,
    )(h0r, pos2, x, S, wqkvT, bqkv2, woT, bo2, ln1g2, ln1b2,
      w1T, b12, w2T, b22, ln2g2, ln2b2, G2, Bc2, whT, bh2)

    return logits


# speedup vs baseline: 2.7886x; 1.0265x over previous
import functools
import math

import jax
import jax.numpy as jnp
from jax import lax
from jax.experimental import pallas as pl
from jax.experimental.pallas import tpu as pltpu

_EPS = 1e-5
_NHEADS = 8
_INTERPRET = False


def _table_prep_kernel(wc_ref, g_ref, b_ref, tc_ref, gacc_ref, bacc_ref):
    i = pl.program_id(0)
    VB = wc_ref.shape[1]
    CD = wc_ref.shape[0]
    t = jnp.transpose(wc_ref[...])
    tc = t * g_ref[...]
    tc_ref[...] = tc.reshape(VB, 1, CD)
    gp = jnp.sum(tc, axis=0, keepdims=True)
    bp = jnp.sum(t * b_ref[...], axis=0, keepdims=True)

    @pl.when(i == 0)
    def _():
        gacc_ref[...] = jnp.zeros_like(gacc_ref)
        bacc_ref[...] = jnp.zeros_like(bacc_ref)

    gacc_ref[...] += gp
    bacc_ref[...] += bp


def _gather_kernel(x_smem, embed_any, tc_ref, h0_ref, s_ref, sem):
    per = pl.num_programs(1)
    b = pl.program_id(0) * per + pl.program_id(1)
    L = x_smem.shape[1]
    CD = tc_ref.shape[2]

    copies = []
    for t in range(L):
        tok = x_smem[b, t]
        cp = pltpu.make_async_copy(
            embed_any.at[pl.ds(tok, 1), :], h0_ref.at[0, t], sem)
        cp.start()
        copies.append(cp)

    accs = [jnp.zeros((1, CD), jnp.float32) for _ in range(8)]
    for t in range(L):
        tok = x_smem[b, t]
        accs[t % 8] = accs[t % 8] + tc_ref[tok]
    acc = ((accs[0] + accs[1]) + (accs[2] + accs[3])) + (
        (accs[4] + accs[5]) + (accs[6] + accs[7]))
    s_ref[0] = acc

    for cp in copies:
        cp.wait()


def _ln(v, g, b):
    m = jnp.mean(v, axis=-1, keepdims=True)
    xc = v - m
    var = jnp.mean(xc * xc, axis=-1, keepdims=True)
    return xc * lax.rsqrt(var + _EPS) * g + b


def _gelu_exact(v):
    return 0.5 * v * (1.0 + lax.erf(v * (1.0 / math.sqrt(2.0))))


def _encoder_kernel(V, h0_any, pos_ref, x_ref, s_in_ref,
                    wqkv_ref, bqkv_ref, wo_ref, bo_ref, ln1g_ref, ln1b_ref,
                    w1_ref, b1_ref, w2_ref, b2_ref, ln2g_ref, ln2b_ref,
                    tc0_ref, g_ref, bc2_ref, wht_ref, bh_ref,
                    out_ref, h_s, qkv_s, sc_s, pb_s, o_s, ff_s, sem):
    c = pl.program_id(0)
    l = pl.program_id(1)
    NL = pl.num_programs(1)
    S_half, L = x_ref.shape
    D = pos_ref.shape[1]
    DH = D // _NHEADS
    CD = s_in_ref.shape[2]
    bf = jnp.bfloat16
    f32 = jnp.float32
    scale = 1.0 / math.sqrt(DH)

    @pl.when(l == 0)
    def _init():
        cps = []
        for s in range(S_half):
            cp = pltpu.make_async_copy(
                h0_any.at[c * S_half + s, :, 0],
                h_s.at[pl.ds(s * L, L), :], sem)
            cp.start()
            cps.append(cp)
        for cp in cps:
            cp.wait()
        for s in range(S_half):
            h_s[s * L:(s + 1) * L, :] = h_s[s * L:(s + 1) * L, :] + pos_ref[...]

    wqkv = wqkv_ref[0]
    wo = wo_ref[0]
    w1 = w1_ref[0]
    w2 = w2_ref[0]

    def seq_body(s, carry):
        off = pl.multiple_of(s * L, L)
        hs = h_s[pl.ds(off, L), :]
        hsb = hs.astype(bf)
        for j in range(3):
            part = lax.dot_general(
                hsb, wqkv[j * D:(j + 1) * D, :], (((1,), (1,)), ((), ())),
                preferred_element_type=f32) + bqkv_ref[0, :, j * D:(j + 1) * D]
            if j == 0:
                part = part * scale
            qkv_s[:, j * D:(j + 1) * D] = part.astype(bf)
        for hh in range(_NHEADS):
            sc = sc_s.at[hh % 2]
            pb = pb_s.at[hh % 2]
            q = qkv_s[:, hh * DH:(hh + 1) * DH]
            k = qkv_s[:, D + hh * DH:D + (hh + 1) * DH]
            sc[...] = lax.dot_general(
                q, k, (((1,), (1,)), ((), ())), preferred_element_type=f32)
            for rc in range(L // 128):
                blk = sc[rc * 128:(rc + 1) * 128, :]
                mx = jnp.max(blk, axis=-1, keepdims=True)
                e = jnp.exp(blk - mx)
                sm = jnp.sum(e, axis=-1, keepdims=True)
                pb[rc * 128:(rc + 1) * 128, :] = (e * (1.0 / sm)).astype(bf)
            v = qkv_s[:, 2 * D + hh * DH:2 * D + (hh + 1) * DH]
            o_s[:, hh * DH:(hh + 1) * DH] = lax.dot_general(
                pb[...], v, (((1,), (0,)), ((), ())),
                preferred_element_type=f32).astype(bf)
        attn = lax.dot_general(
            o_s[...], wo, (((1,), (1,)), ((), ())),
            preferred_element_type=f32) + bo_ref[0]
        h1 = _ln(hs + attn, ln1g_ref[0], ln1b_ref[0])
        h1b = h1.astype(bf)
        for cc in range(4):
            part = lax.dot_general(
                h1b, w1[cc * D:(cc + 1) * D, :], (((1,), (1,)), ((), ())),
                preferred_element_type=f32) + b1_ref[0, :, cc * D:(cc + 1) * D]
            ff_s[:, cc * D:(cc + 1) * D] = _gelu_exact(part).astype(bf)
        ff2 = lax.dot_general(
            ff_s[...], w2, (((1,), (1,)), ((), ())),
            preferred_element_type=f32) + b2_ref[0]
        h_s[pl.ds(off, L), :] = _ln(h1 + ff2, ln2g_ref[0], ln2b_ref[0])
        return carry

    lax.fori_loop(0, S_half, seq_body, 0)

    @pl.when(l == NL - 1)
    def _head():
        ctxs = []
        for s in range(S_half):
            hs = h_s[s * L:(s + 1) * L, :]
            ctxs.append(jnp.mean(hs, axis=0, keepdims=True))
        ctx = jnp.concatenate(ctxs, axis=0)

        xb = x_ref[...]
        iota = lax.broadcasted_iota(jnp.int32, xb.shape, 1)
        xs = jnp.where(xb == 0, -(iota + 1), xb)
        nb = jnp.sum((xb != 0).astype(f32), axis=-1, keepdims=True)
        npad = float(L) - nb
        cnt = jnp.zeros(xs.shape, f32)
        for t in range(L):
            col = xs[:, t:t + 1]
            cnt = cnt + (xs == col).astype(f32)
        pair = jnp.sum(cnt, axis=-1, keepdims=True)
        sumsq = pair - npad
        m = nb * (1.0 / V)
        var = sumsq * (1.0 / V) - m * m
        r = lax.rsqrt(var + _EPS)
        S = s_in_ref[...].reshape(S_half, CD) - npad * tc0_ref[...]
        cf = jnp.maximum(r * S - (r * m) * g_ref[...] + bc2_ref[...], 0.0)
        feat = jnp.concatenate([ctx, cf], axis=-1)
        out_ref[...] = jnp.dot(feat, wht_ref[...],
                               preferred_element_type=f32) + bh_ref[...]


def kernel(x, embed, pos, Wqkv, bqkv, Wo, bo, ln1_g, ln1_b, W1, b1, W2, b2,
           ln2_g, ln2_b, cnt_ln_g, cnt_ln_b, Wc, bc, Wh, bh):
    B, L = x.shape
    V, D = embed.shape
    NL = Wqkv.shape[0]
    CD = Wc.shape[0]
    C = Wh.shape[0]
    S_half = B // 2
    bf = jnp.bfloat16
    f32 = jnp.float32

    wqkvT = Wqkv.astype(bf)
    woT = Wo.astype(bf)
    w1T = W1.astype(bf)
    w2T = W2.astype(bf)
    bqkv2 = bqkv[:, None, :]
    bo2 = bo[:, None, :]
    b12 = b1[:, None, :]
    b22 = b2[:, None, :]
    ln1g2 = ln1_g[:, None, :]
    ln1b2 = ln1_b[:, None, :]
    ln2g2 = ln2_g[:, None, :]
    ln2b2 = ln2_b[:, None, :]
    pos2 = pos[0]
    whT = Wh.T
    bh2 = bh[None, :]
    Vp = V
    Tc = (Wc * cnt_ln_g[None, :]).T[:, None, :]
    tc0 = (Wc[:, 0] * cnt_ln_g[0])[None, :]
    G2 = (Wc @ cnt_ln_g)[None, :]
    Bc2 = (Wc @ cnt_ln_b + bc)[None, :]

    h0, S = pl.pallas_call(
        _gather_kernel,
        grid=(2, S_half),
        in_specs=[
            pl.BlockSpec(memory_space=pltpu.SMEM),
            pl.BlockSpec(memory_space=pl.ANY),
            pl.BlockSpec((Vp, 1, CD), lambda c, i: (0, 0, 0)),
        ],
        out_specs=[
            pl.BlockSpec((1, L, 1, D), lambda c, i, _per=S_half: (c * _per + i, 0, 0, 0)),
            pl.BlockSpec((1, 1, CD), lambda c, i, _per=S_half: (c * _per + i, 0, 0)),
        ],
        out_shape=[
            jax.ShapeDtypeStruct((B, L, 1, D), f32),
            jax.ShapeDtypeStruct((B, 1, CD), f32),
        ],
        scratch_shapes=[pltpu.SemaphoreType.DMA],
        compiler_params=pltpu.CompilerParams(
            dimension_semantics=("parallel", "arbitrary"),
            vmem_limit_bytes=44 * 1024 * 1024,
        ),
        name="embed_gather",
        interpret=_INTERPRET,
    )(x, embed, Tc)


    logits = pl.pallas_call(
        functools.partial(_encoder_kernel, float(V)),
        grid=(2, NL),
        in_specs=[
            pl.BlockSpec(memory_space=pl.ANY),
            pl.BlockSpec((L, D), lambda c, l: (0, 0)),
            pl.BlockSpec((S_half, L), lambda c, l: (c, 0)),
            pl.BlockSpec((S_half, 1, CD), lambda c, l: (c, 0, 0)),
            pl.BlockSpec((1, 3 * D, D), lambda c, l: (l, 0, 0)),
            pl.BlockSpec((1, 1, 3 * D), lambda c, l: (l, 0, 0)),
            pl.BlockSpec((1, D, D), lambda c, l: (l, 0, 0)),
            pl.BlockSpec((1, 1, D), lambda c, l: (l, 0, 0)),
            pl.BlockSpec((1, 1, D), lambda c, l: (l, 0, 0)),
            pl.BlockSpec((1, 1, D), lambda c, l: (l, 0, 0)),
            pl.BlockSpec((1, 4 * D, D), lambda c, l: (l, 0, 0)),
            pl.BlockSpec((1, 1, 4 * D), lambda c, l: (l, 0, 0)),
            pl.BlockSpec((1, D, 4 * D), lambda c, l: (l, 0, 0)),
            pl.BlockSpec((1, 1, D), lambda c, l: (l, 0, 0)),
            pl.BlockSpec((1, 1, D), lambda c, l: (l, 0, 0)),
            pl.BlockSpec((1, 1, D), lambda c, l: (l, 0, 0)),
            pl.BlockSpec((1, CD), lambda c, l: (0, 0)),
            pl.BlockSpec((1, CD), lambda c, l: (0, 0)),
            pl.BlockSpec((1, CD), lambda c, l: (0, 0)),
            pl.BlockSpec((D + CD, C), lambda c, l: (0, 0)),
            pl.BlockSpec((1, C), lambda c, l: (0, 0)),
        ],
        out_specs=pl.BlockSpec((S_half, C), lambda c, l: (c, 0)),
        out_shape=jax.ShapeDtypeStruct((B, C), f32),
        scratch_shapes=[
            pltpu.VMEM((S_half * L, D), f32),
            pltpu.VMEM((L, 3 * D), bf),
            pltpu.VMEM((2, L, L), f32),
            pltpu.VMEM((2, L, L), bf),
            pltpu.VMEM((L, D), bf),
            pltpu.VMEM((L, 4 * D), bf),
            pltpu.SemaphoreType.DMA,
        ],
        compiler_params=pltpu.CompilerParams(
            dimension_semantics=("parallel", "arbitrary"),
            vmem_limit_bytes=50 * 1024 * 1024,
        ),
        name="encoder_head",
        interpret=_INTERPRET,
    )(h0, pos2, x, S, wqkvT, bqkv2, woT, bo2, ln1g2, ln1b2,
      w1T, b12, w2T, b22, ln2g2, ln2b2, tc0, G2, Bc2, whT, bh2)

    return logits
